# baseline (speedup 1.0000x reference)
import functools
import math

import jax
import jax.numpy as jnp
from jax import lax
from jax.experimental import pallas as pl
from jax.experimental.pallas import tpu as pltpu
from jax.experimental.pallas import tpu_sc as plsc

D_MODEL = 64
SCALE = math.sqrt(D_MODEL)

_info = plsc.get_sparse_core_info()
_NC, _NS, _L = _info.num_cores, _info.num_subcores, _info.num_lanes
_NW = _NC * _NS

_NI = 16384
_NJ = 50
_B = _NI * _NJ
_IB = _NI // _NW
_NQ = _IB // 128
_NTASK = _NJ * _NQ
_IDXW = _IB * _NJ
_NB = 2


@functools.partial(
    pl.kernel,
    out_type=jax.ShapeDtypeStruct((_NJ, 8, _NI // 128, 1024), jnp.float32),
    mesh=plsc.VectorSubcoreMesh(core_axis_name="c", subcore_axis_name="s"),
    scratch_types=[
        pltpu.VMEM((_IDXW,), jnp.int32),
        pltpu.VMEM((_NB, 128), jnp.int32),
        pltpu.VMEM((_NB, 128, D_MODEL), jnp.float32),
        pltpu.VMEM((_NB, 8192), jnp.float32),
    ]
    + [pltpu.SemaphoreType.DMA] * (2 * _NB),
    compiler_params=pltpu.CompilerParams(
        use_tc_tiling_on_sc=False, needs_layout_passes=False
    ),
)
def _emb_kernel(x_hbm, lut_hbm, out_hbm, idx_v, idxc, rows, obuf, *sems):
    gsems, osems = sems[:_NB], sems[_NB:]
    wid = lax.axis_index("s") * _NC + lax.axis_index("c")
    pltpu.sync_copy(x_hbm.at[pl.ds(wid * _IDXW, _IDXW)], idx_v)

    def jq(t):
        return lax.rem(t, _NJ), lax.div(t, _NJ)

    def build_and_fire(t, b):
        j, q = jq(t)
        base = q * (128 * _NJ) + j
        for g in range(8):
            pos = base + (g * 16 * _NJ) + lax.iota(jnp.int32, 16) * _NJ
            idxc[b, pl.ds(g * 16, 16)] = plsc.load_gather(idx_v, [pos])
        pltpu.async_copy(lut_hbm.at[idxc.at[b]], rows.at[b], gsems[b])

    for b in range(_NB):
        build_and_fire(b, b)

    @pl.loop(0, _NTASK // _NB)
    def _task(tt):
        for b in range(_NB):
            t = tt * _NB + b
            j, q = jq(t)
            it = wid * _NQ + q
            pltpu.make_async_copy(
                lut_hbm.at[pl.ds(0, 128)], rows.at[b], gsems[b]
            ).wait()
            @pl.when(t >= _NB)
            def _drain():
                for kt in range(8):
                    pltpu.make_async_copy(
                        obuf.at[b, pl.ds(0, 1024)],
                        out_hbm.at[0, 0, 0],
                        osems[b],
                    ).wait()

            for g in range(8):
                rowv = lax.iota(jnp.int32, 16) + g * 16

                @plsc.parallel_loop(0, 8, unroll=2)
                def _kt(kt):
                    for ks in range(8):
                        c = kt * 8 + ks
                        colv = jnp.full((16,), 0, jnp.int32) + c
                        vals = plsc.load_gather(rows.at[b], [rowv, colv])
                        obuf[b, pl.ds(kt * 1024 + ks * 128 + g * 16, 16)] = (
                            vals * SCALE
                        )

            @pl.when(t + _NB < _NTASK)
            def _next():
                build_and_fire(t + _NB, b)

            for kt in range(8):
                pltpu.async_copy(
                    obuf.at[b, pl.ds(kt * 1024, 1024)],
                    out_hbm.at[j, kt, it],
                    osems[b],
                )

    for b in range(_NB):
        for kt in range(8):
            pltpu.make_async_copy(
                obuf.at[b, pl.ds(0, 1024)], out_hbm.at[0, 0, 0], osems[b]
            ).wait()


def kernel(x, lut):
    x_flat = x.reshape(-1).astype(jnp.int32)
    out5 = _emb_kernel(x_flat, lut)
    out5 = out5.reshape(_NJ, 8, _NI // 128, 8, 128)
    return out5.transpose(2, 4, 0, 1, 3).reshape(_NI, _NJ, D_MODEL)


# speedup vs baseline: 1.6640x; 1.6640x over previous
import functools
import math

import jax
import jax.numpy as jnp
from jax import lax
from jax.experimental import pallas as pl
from jax.experimental.pallas import tpu as pltpu
from jax.experimental.pallas import tpu_sc as plsc

D_MODEL = 64
SCALE = math.sqrt(D_MODEL)

_info = plsc.get_sparse_core_info()
_NC, _NS, _L = _info.num_cores, _info.num_subcores, _info.num_lanes
_NW = _NC * _NS

_NI = 16384
_NJ = 50
_B = _NI * _NJ
_IB = _NI // _NW
_NQ = _IB // 128
_NTASK = _NJ * _NQ
_IDXW = _IB * _NJ
_NB = 2
_DPAD = 65


@functools.partial(
    pl.kernel,
    out_type=jax.ShapeDtypeStruct((_NJ, 8, _NI // 128, 1024), jnp.float32),
    mesh=plsc.VectorSubcoreMesh(core_axis_name="c", subcore_axis_name="s"),
    scratch_types=[
        pltpu.VMEM((_IDXW,), jnp.int32),
        pltpu.VMEM((_NB, 128), jnp.int32),
        pltpu.VMEM((_NB, 128, D_MODEL), jnp.float32),
        pltpu.VMEM((_NB, 128, _DPAD), jnp.float32),
        pltpu.VMEM((_NB, 8192), jnp.float32),
    ]
    + [pltpu.SemaphoreType.DMA] * (2 * _NB),
    compiler_params=pltpu.CompilerParams(
        use_tc_tiling_on_sc=False, needs_layout_passes=False
    ),
)
def _emb_kernel(x_hbm, lut_hbm, out_hbm, idx_v, idxc, rows, rows65, obuf, *sems):
    gsems, osems = sems[:_NB], sems[_NB:]
    wid = lax.axis_index("s") * _NC + lax.axis_index("c")
    pltpu.sync_copy(x_hbm.at[pl.ds(wid * _IDXW, _IDXW)], idx_v)

    iota = lax.iota(jnp.int32, 16)
    iota_nj = iota * _NJ

    def jq(t):
        return lax.rem(t, _NJ), lax.div(t, _NJ)

    def build_and_fire(t, b):
        j, q = jq(t)
        base = q * (128 * _NJ) + j
        for g in range(8):
            pos = base + (g * 16 * _NJ) + iota_nj
            idxc[b, pl.ds(g * 16, 16)] = plsc.load_gather(idx_v, [pos])
        pltpu.async_copy(lut_hbm.at[idxc.at[b]], rows.at[b], gsems[b])

    for b in range(_NB):
        build_and_fire(b, b)

    @pl.loop(0, _NTASK // _NB)
    def _task(tt):
        for b in range(_NB):
            t = tt * _NB + b
            j, q = jq(t)
            it = wid * _NQ + q
            pltpu.make_async_copy(
                lut_hbm.at[pl.ds(0, 128)], rows.at[b], gsems[b]
            ).wait()
            @pl.when(t >= _NB)
            def _drain():
                for kt in range(8):
                    pltpu.make_async_copy(
                        obuf.at[b, pl.ds(0, 1024)],
                        out_hbm.at[0, 0, 0],
                        osems[b],
                    ).wait()

            @plsc.parallel_loop(0, 128, unroll=4)
            def _restride(r):
                for sseg in range(4):
                    sl = pl.ds(sseg * 16, 16)
                    rows65[b, r, sl] = rows[b, r, sl]

            @plsc.parallel_loop(0, D_MODEL, unroll=2)
            def _col(c):
                cbase = (
                    lax.shift_left(lax.shift_right_logical(c, 3), 10)
                    + lax.shift_left(lax.bitwise_and(c, 7), 7)
                )
                colv = iota * 0 + c
                for g in range(8):
                    vals = plsc.load_gather(
                        rows65.at[b], [iota + g * 16, colv]
                    )
                    obuf[b, pl.ds(cbase + g * 16, 16)] = vals * SCALE

            @pl.when(t + _NB < _NTASK)
            def _next():
                build_and_fire(t + _NB, b)

            for kt in range(8):
                pltpu.async_copy(
                    obuf.at[b, pl.ds(kt * 1024, 1024)],
                    out_hbm.at[j, kt, it],
                    osems[b],
                )

    for b in range(_NB):
        for kt in range(8):
            pltpu.make_async_copy(
                obuf.at[b, pl.ds(0, 1024)], out_hbm.at[0, 0, 0], osems[b]
            ).wait()


def kernel(x, lut):
    x_flat = x.reshape(-1).astype(jnp.int32)
    out5 = _emb_kernel(x_flat, lut)
    out5 = out5.reshape(_NJ, 8, _NI // 128, 8, 128)
    return out5.transpose(2, 4, 0, 1, 3).reshape(_NI, _NJ, D_MODEL)


# speedup vs baseline: 1.9938x; 1.1982x over previous
import functools
import math

import jax
import jax.numpy as jnp
from jax import lax
from jax.experimental import pallas as pl
from jax.experimental.pallas import tpu as pltpu
from jax.experimental.pallas import tpu_sc as plsc

D_MODEL = 64
SCALE = math.sqrt(D_MODEL)

_info = plsc.get_sparse_core_info()
_NC, _NS, _L = _info.num_cores, _info.num_subcores, _info.num_lanes
_NW = _NC * _NS

_NI = 16384
_NJ = 50
_B = _NI * _NJ
_IB = _NI // _NW
_NQ = _IB // 128
_NTASK = _NJ * _NQ
_IDXW = _IB * _NJ
_NB = 2
_DPAD = 65

_FW = 1024
_FH = 500736
_FD = 499712


def _fmt_body(in_lo, in_hi, outb):
    lo = jnp.transpose(in_lo[...], (1, 0))
    hi = jnp.transpose(in_hi[...], (1, 0))
    outb[...] = jnp.concatenate([lo, hi], axis=1)


_fmt = pl.pallas_call(
    _fmt_body,
    out_shape=jax.ShapeDtypeStruct((_FH, 128), jnp.float32),
    grid=(_FH // _FW,),
    in_specs=[
        pl.BlockSpec((64, _FW), lambda i: (0, i)),
        pl.BlockSpec((64, _FW), lambda i: (0, i + _FD // _FW)),
    ],
    out_specs=pl.BlockSpec((_FW, 128), lambda i: (i, 0)),
)


@functools.partial(
    pl.kernel,
    out_type=jax.ShapeDtypeStruct((_NJ, 8, _NI // 128, 1024), jnp.float32),
    mesh=plsc.VectorSubcoreMesh(core_axis_name="c", subcore_axis_name="s"),
    scratch_types=[
        pltpu.VMEM((_IDXW,), jnp.int32),
        pltpu.VMEM((_NB, 128), jnp.int32),
        pltpu.VMEM((_NB, 128, D_MODEL), jnp.float32),
        pltpu.VMEM((_NB, 128, _DPAD), jnp.float32),
        pltpu.VMEM((_NB, 8192), jnp.float32),
    ]
    + [pltpu.SemaphoreType.DMA] * (2 * _NB),
    compiler_params=pltpu.CompilerParams(
        use_tc_tiling_on_sc=False, needs_layout_passes=False
    ),
)
def _emb_kernel(x_hbm, lut_hbm, out_hbm, idx_v, idxc, rows, rows65, obuf, *sems):
    gsems, osems = sems[:_NB], sems[_NB:]
    wid = lax.axis_index("s") * _NC + lax.axis_index("c")
    pltpu.sync_copy(x_hbm.at[pl.ds(wid * _IDXW, _IDXW)], idx_v)

    iota = lax.iota(jnp.int32, 16)
    iota_nj = iota * _NJ

    def jq(t):
        return lax.rem(t, _NJ), lax.div(t, _NJ)

    def build_and_fire(t, b):
        j, q = jq(t)
        base = q * (128 * _NJ) + j
        for g in range(8):
            pos = base + (g * 16 * _NJ) + iota_nj
            v = plsc.load_gather(idx_v, [pos])
            idxc[b, pl.ds(g * 16, 16)] = jnp.where(
                v >= _FH, (v - _FD) * 2 + 1, v * 2
            )
        pltpu.async_copy(lut_hbm.at[idxc.at[b]], rows.at[b], gsems[b])

    for b in range(_NB):
        build_and_fire(b, b)

    @pl.loop(0, _NTASK // _NB)
    def _task(tt):
        for b in range(_NB):
            t = tt * _NB + b
            j, q = jq(t)
            it = wid * _NQ + q
            pltpu.make_async_copy(
                lut_hbm.at[pl.ds(0, 128)], rows.at[b], gsems[b]
            ).wait()
            @pl.when(t >= _NB)
            def _drain():
                for kt in range(8):
                    pltpu.make_async_copy(
                        obuf.at[b, pl.ds(0, 1024)],
                        out_hbm.at[0, 0, 0],
                        osems[b],
                    ).wait()

            @plsc.parallel_loop(0, 128, unroll=4)
            def _restride(r):
                for sseg in range(4):
                    sl = pl.ds(sseg * 16, 16)
                    rows65[b, r, sl] = rows[b, r, sl]

            @plsc.parallel_loop(0, D_MODEL, unroll=2)
            def _col(c):
                cbase = (
                    lax.shift_left(lax.shift_right_logical(c, 3), 10)
                    + lax.shift_left(lax.bitwise_and(c, 7), 7)
                )
                colv = iota * 0 + c
                for g in range(8):
                    vals = plsc.load_gather(
                        rows65.at[b], [iota + g * 16, colv]
                    )
                    obuf[b, pl.ds(cbase + g * 16, 16)] = vals * SCALE

            @pl.when(t + _NB < _NTASK)
            def _next():
                build_and_fire(t + _NB, b)

            for kt in range(8):
                pltpu.async_copy(
                    obuf.at[b, pl.ds(kt * 1024, 1024)],
                    out_hbm.at[j, kt, it],
                    osems[b],
                )

    for b in range(_NB):
        for kt in range(8):
            pltpu.make_async_copy(
                obuf.at[b, pl.ds(0, 1024)], out_hbm.at[0, 0, 0], osems[b]
            ).wait()


def kernel(x, lut):
    x_flat = x.reshape(-1).astype(jnp.int32)
    lut_lin = _fmt(lut.T, lut.T).reshape(2 * _FH, D_MODEL)
    out5 = _emb_kernel(x_flat, lut_lin)
    out5 = out5.reshape(_NJ, 8, _NI // 128, 8, 128)
    return out5.transpose(2, 4, 0, 1, 3).reshape(_NI, _NJ, D_MODEL)


# speedup vs baseline: 2.7539x; 1.3813x over previous
import functools
import math

import jax
import jax.numpy as jnp
from jax import lax
from jax.experimental import pallas as pl
from jax.experimental.pallas import tpu as pltpu
from jax.experimental.pallas import tpu_sc as plsc

D_MODEL = 64
SCALE = math.sqrt(D_MODEL)

_info = plsc.get_sparse_core_info()
_NC, _NS, _L = _info.num_cores, _info.num_subcores, _info.num_lanes
_NW = _NC * _NS

_NI = 16384
_NJ = 50
_B = _NI * _NJ
_IB = _NI // _NW
_NQ = _IB // 128
_NTASK = _NJ * _NQ
_IDXW = _IB * _NJ
_NB = 2
_DPAD = 65

_FW = 4096
_FH = 503808
_FD = 499712


def _fmt_body(in_lo, in_hi, outb):
    lo = jnp.transpose(in_lo[...], (1, 0))
    hi = jnp.transpose(in_hi[...], (1, 0))
    outb[...] = jnp.concatenate([lo, hi], axis=1)


_fmt = pl.pallas_call(
    _fmt_body,
    out_shape=jax.ShapeDtypeStruct((_FH, 128), jnp.float32),
    grid=(_FH // _FW,),
    in_specs=[
        pl.BlockSpec((64, _FW), lambda i: (0, i)),
        pl.BlockSpec((64, _FW), lambda i: (0, i + _FD // _FW)),
    ],
    out_specs=pl.BlockSpec((_FW, 128), lambda i: (i, 0)),
)


@functools.partial(
    pl.kernel,
    out_type=jax.ShapeDtypeStruct((_NJ, 8, _NI // 128, 1024), jnp.float32),
    mesh=plsc.VectorSubcoreMesh(core_axis_name="c", subcore_axis_name="s"),
    scratch_types=[
        pltpu.VMEM((_IDXW,), jnp.int32),
        pltpu.VMEM((_NB, 128), jnp.int32),
        pltpu.VMEM((_NB, 128, D_MODEL), jnp.float32),
        pltpu.VMEM((_NB, 128, _DPAD), jnp.float32),
        pltpu.VMEM((_NB, 8192), jnp.float32),
    ]
    + [pltpu.SemaphoreType.DMA] * (2 * _NB),
    compiler_params=pltpu.CompilerParams(
        use_tc_tiling_on_sc=False, needs_layout_passes=False
    ),
)
def _emb_kernel(x_hbm, lut_hbm, out_hbm, idx_v, idxc, rows, rows65, obuf, *sems):
    gsems, osems = sems[:_NB], sems[_NB:]
    wid = lax.axis_index("s") * _NC + lax.axis_index("c")
    pltpu.sync_copy(x_hbm.at[pl.ds(wid * _IDXW, _IDXW)], idx_v)

    iota = lax.iota(jnp.int32, 16)
    iota_nj = iota * _NJ

    def jq(t):
        return lax.rem(t, _NJ), lax.div(t, _NJ)

    def build_and_fire(t, b):
        j, q = jq(t)
        base = q * (128 * _NJ) + j
        for g in range(8):
            pos = base + (g * 16 * _NJ) + iota_nj
            v = plsc.load_gather(idx_v, [pos])
            idxc[b, pl.ds(g * 16, 16)] = jnp.where(
                v >= _FH, (v - _FD) * 2 + 1, v * 2
            )
        pltpu.async_copy(lut_hbm.at[idxc.at[b]], rows.at[b], gsems[b])

    for b in range(_NB):
        build_and_fire(b, b)

    @pl.loop(0, _NTASK // _NB)
    def _task(tt):
        for b in range(_NB):
            t = tt * _NB + b
            j, q = jq(t)
            it = wid * _NQ + q
            pltpu.make_async_copy(
                lut_hbm.at[pl.ds(0, 128)], rows.at[b], gsems[b]
            ).wait()
            @pl.when(t >= _NB)
            def _drain():
                for kt in range(8):
                    pltpu.make_async_copy(
                        obuf.at[b, pl.ds(0, 1024)],
                        out_hbm.at[0, 0, 0],
                        osems[b],
                    ).wait()

            @plsc.parallel_loop(0, 128, unroll=4)
            def _restride(r):
                for sseg in range(4):
                    sl = pl.ds(sseg * 16, 16)
                    rows65[b, r, sl] = rows[b, r, sl]

            @plsc.parallel_loop(0, D_MODEL, unroll=2)
            def _col(c):
                cbase = (
                    lax.shift_left(lax.shift_right_logical(c, 3), 10)
                    + lax.shift_left(lax.bitwise_and(c, 7), 7)
                )
                colv = iota * 0 + c
                for g in range(8):
                    vals = plsc.load_gather(
                        rows65.at[b], [iota + g * 16, colv]
                    )
                    obuf[b, pl.ds(cbase + g * 16, 16)] = vals * SCALE

            @pl.when(t + _NB < _NTASK)
            def _next():
                build_and_fire(t + _NB, b)

            for kt in range(8):
                pltpu.async_copy(
                    obuf.at[b, pl.ds(kt * 1024, 1024)],
                    out_hbm.at[j, kt, it],
                    osems[b],
                )

    for b in range(_NB):
        for kt in range(8):
            pltpu.make_async_copy(
                obuf.at[b, pl.ds(0, 1024)], out_hbm.at[0, 0, 0], osems[b]
            ).wait()


def kernel(x, lut):
    x_flat = x.reshape(-1).astype(jnp.int32)
    lut_lin = _fmt(lut.T, lut.T).reshape(2 * _FH, D_MODEL)
    out5 = _emb_kernel(x_flat, lut_lin)
    out5 = out5.reshape(_NJ, 8, _NI // 128, 8, 128)
    return out5.transpose(2, 4, 0, 1, 3).reshape(_NI, _NJ, D_MODEL)


# speedup vs baseline: 2.9414x; 1.0681x over previous
import functools
import math

import jax
import jax.numpy as jnp
from jax import lax
from jax.experimental import pallas as pl
from jax.experimental.pallas import tpu as pltpu
from jax.experimental.pallas import tpu_sc as plsc

D_MODEL = 64
SCALE = math.sqrt(D_MODEL)

_info = plsc.get_sparse_core_info()
_NC, _NS, _L = _info.num_cores, _info.num_subcores, _info.num_lanes
_NW = _NC * _NS

_NI = 16384
_NJ = 50
_B = _NI * _NJ
_IB = _NI // _NW
_NQ = _IB // 128
_NTASK = _NJ * _NQ
_IDXW = _IB * _NJ
_NB = 2
_DPAD = 65

_FW = 8192
_FH = 507904
_FD = 499712


def _fmt_body(in_lo, in_hi, outb):
    lo = jnp.transpose(in_lo[...], (1, 0))
    hi = jnp.transpose(in_hi[...], (1, 0))
    outb[...] = jnp.concatenate([lo, hi], axis=1)


_fmt = pl.pallas_call(
    _fmt_body,
    out_shape=jax.ShapeDtypeStruct((_FH, 128), jnp.float32),
    grid=(_FH // _FW,),
    in_specs=[
        pl.BlockSpec((64, _FW), lambda i: (0, i)),
        pl.BlockSpec((64, _FW), lambda i: (0, i + _FD // _FW)),
    ],
    out_specs=pl.BlockSpec((_FW, 128), lambda i: (i, 0)),
)


@functools.partial(
    pl.kernel,
    out_type=jax.ShapeDtypeStruct((_NJ, 8, _NI // 128, 1024), jnp.float32),
    mesh=plsc.VectorSubcoreMesh(core_axis_name="c", subcore_axis_name="s"),
    scratch_types=[
        pltpu.VMEM((_IDXW,), jnp.int32),
        pltpu.VMEM((_NB, 128), jnp.int32),
        pltpu.VMEM((_NB, 128, D_MODEL), jnp.float32),
        pltpu.VMEM((_NB, 128, _DPAD), jnp.float32),
        pltpu.VMEM((_NB, 8192), jnp.float32),
    ]
    + [pltpu.SemaphoreType.DMA] * (2 * _NB),
    compiler_params=pltpu.CompilerParams(
        use_tc_tiling_on_sc=False, needs_layout_passes=False
    ),
)
def _emb_kernel(x_hbm, lut_hbm, out_hbm, idx_v, idxc, rows, rows65, obuf, *sems):
    gsems, osems = sems[:_NB], sems[_NB:]
    wid = lax.axis_index("s") * _NC + lax.axis_index("c")
    pltpu.sync_copy(x_hbm.at[pl.ds(wid * _IDXW, _IDXW)], idx_v)

    iota = lax.iota(jnp.int32, 16)
    iota_nj = iota * _NJ

    def jq(t):
        return lax.rem(t, _NJ), lax.div(t, _NJ)

    def build_and_fire(t, b):
        j, q = jq(t)
        base = q * (128 * _NJ) + j
        for g in range(8):
            pos = base + (g * 16 * _NJ) + iota_nj
            v = plsc.load_gather(idx_v, [pos])
            idxc[b, pl.ds(g * 16, 16)] = jnp.where(
                v >= _FH, (v - _FD) * 2 + 1, v * 2
            )
        pltpu.async_copy(lut_hbm.at[idxc.at[b]], rows.at[b], gsems[b])

    for b in range(_NB):
        build_and_fire(b, b)

    @pl.loop(0, _NTASK // _NB)
    def _task(tt):
        for b in range(_NB):
            t = tt * _NB + b
            j, q = jq(t)
            it = wid * _NQ + q
            pltpu.make_async_copy(
                lut_hbm.at[pl.ds(0, 128)], rows.at[b], gsems[b]
            ).wait()
            @pl.when(t >= _NB)
            def _drain():
                for kt in range(8):
                    pltpu.make_async_copy(
                        obuf.at[b, pl.ds(0, 1024)],
                        out_hbm.at[0, 0, 0],
                        osems[b],
                    ).wait()

            @plsc.parallel_loop(0, 128, unroll=4)
            def _restride(r):
                for sseg in range(4):
                    sl = pl.ds(sseg * 16, 16)
                    rows65[b, r, sl] = rows[b, r, sl]

            @plsc.parallel_loop(0, D_MODEL, unroll=2)
            def _col(c):
                cbase = (
                    lax.shift_left(lax.shift_right_logical(c, 3), 10)
                    + lax.shift_left(lax.bitwise_and(c, 7), 7)
                )
                colv = iota * 0 + c
                for g in range(8):
                    vals = plsc.load_gather(
                        rows65.at[b], [iota + g * 16, colv]
                    )
                    obuf[b, pl.ds(cbase + g * 16, 16)] = vals * SCALE

            @pl.when(t + _NB < _NTASK)
            def _next():
                build_and_fire(t + _NB, b)

            for kt in range(8):
                pltpu.async_copy(
                    obuf.at[b, pl.ds(kt * 1024, 1024)],
                    out_hbm.at[j, kt, it],
                    osems[b],
                )

    for b in range(_NB):
        for kt in range(8):
            pltpu.make_async_copy(
                obuf.at[b, pl.ds(0, 1024)], out_hbm.at[0, 0, 0], osems[b]
            ).wait()


def kernel(x, lut):
    x_flat = x.reshape(-1).astype(jnp.int32)
    lut_lin = _fmt(lut.T, lut.T).reshape(2 * _FH, D_MODEL)
    out5 = _emb_kernel(x_flat, lut_lin)
    out5 = out5.reshape(_NJ, 8, _NI // 128, 8, 128)
    return out5.transpose(2, 4, 0, 1, 3).reshape(_NI, _NJ, D_MODEL)


# speedup vs baseline: 2.9933x; 1.0176x over previous
import functools
import math

import jax
import jax.numpy as jnp
from jax import lax
from jax.experimental import pallas as pl
from jax.experimental.pallas import tpu as pltpu
from jax.experimental.pallas import tpu_sc as plsc

D_MODEL = 64
SCALE = math.sqrt(D_MODEL)

_info = plsc.get_sparse_core_info()
_NC, _NS, _L = _info.num_cores, _info.num_subcores, _info.num_lanes
_NW = _NC * _NS

_NI = 16384
_NJ = 50
_B = _NI * _NJ
_IB = _NI // _NW
_NQ = _IB // 128
_NTASK = _NJ * _NQ
_IDXW = _IB * _NJ
_NB = 2
_DPAD = 65

_FW = 16384
_FH = 524288
_FD = 491520


def _fmt_body(in_lo, in_hi, outb):
    lo = jnp.transpose(in_lo[...], (1, 0))
    hi = jnp.transpose(in_hi[...], (1, 0))
    outb[...] = jnp.concatenate([lo, hi], axis=1)


_fmt = pl.pallas_call(
    _fmt_body,
    out_shape=jax.ShapeDtypeStruct((_FH, 128), jnp.float32),
    grid=(_FH // _FW,),
    in_specs=[
        pl.BlockSpec((64, _FW), lambda i: (0, i)),
        pl.BlockSpec((64, _FW), lambda i: (0, i + _FD // _FW)),
    ],
    out_specs=pl.BlockSpec((_FW, 128), lambda i: (i, 0)),
)


@functools.partial(
    pl.kernel,
    out_type=jax.ShapeDtypeStruct((_NJ, 8, _NI // 128, 1024), jnp.float32),
    mesh=plsc.VectorSubcoreMesh(core_axis_name="c", subcore_axis_name="s"),
    scratch_types=[
        pltpu.VMEM((_IDXW,), jnp.int32),
        pltpu.VMEM((_NB, 128), jnp.int32),
        pltpu.VMEM((_NB, 128, D_MODEL), jnp.float32),
        pltpu.VMEM((_NB, 128, _DPAD), jnp.float32),
        pltpu.VMEM((_NB, 8192), jnp.float32),
    ]
    + [pltpu.SemaphoreType.DMA] * (2 * _NB),
    compiler_params=pltpu.CompilerParams(
        use_tc_tiling_on_sc=False, needs_layout_passes=False
    ),
)
def _emb_kernel(x_hbm, lut_hbm, out_hbm, idx_v, idxc, rows, rows65, obuf, *sems):
    gsems, osems = sems[:_NB], sems[_NB:]
    wid = lax.axis_index("s") * _NC + lax.axis_index("c")
    pltpu.sync_copy(x_hbm.at[pl.ds(wid * _IDXW, _IDXW)], idx_v)

    iota = lax.iota(jnp.int32, 16)
    iota_nj = iota * _NJ

    def jq(t):
        return lax.rem(t, _NJ), lax.div(t, _NJ)

    def build_and_fire(t, b):
        j, q = jq(t)
        base = q * (128 * _NJ) + j
        for g in range(8):
            pos = base + (g * 16 * _NJ) + iota_nj
            v = plsc.load_gather(idx_v, [pos])
            idxc[b, pl.ds(g * 16, 16)] = jnp.where(
                v >= _FH, (v - _FD) * 2 + 1, v * 2
            )
        pltpu.async_copy(lut_hbm.at[idxc.at[b]], rows.at[b], gsems[b])

    for b in range(_NB):
        build_and_fire(b, b)

    @pl.loop(0, _NTASK // _NB)
    def _task(tt):
        for b in range(_NB):
            t = tt * _NB + b
            j, q = jq(t)
            it = wid * _NQ + q
            pltpu.make_async_copy(
                lut_hbm.at[pl.ds(0, 128)], rows.at[b], gsems[b]
            ).wait()
            @pl.when(t >= _NB)
            def _drain():
                for kt in range(8):
                    pltpu.make_async_copy(
                        obuf.at[b, pl.ds(0, 1024)],
                        out_hbm.at[0, 0, 0],
                        osems[b],
                    ).wait()

            @plsc.parallel_loop(0, 128, unroll=4)
            def _restride(r):
                for sseg in range(4):
                    sl = pl.ds(sseg * 16, 16)
                    rows65[b, r, sl] = rows[b, r, sl]

            @plsc.parallel_loop(0, D_MODEL, unroll=2)
            def _col(c):
                cbase = (
                    lax.shift_left(lax.shift_right_logical(c, 3), 10)
                    + lax.shift_left(lax.bitwise_and(c, 7), 7)
                )
                colv = iota * 0 + c
                for g in range(8):
                    vals = plsc.load_gather(
                        rows65.at[b], [iota + g * 16, colv]
                    )
                    obuf[b, pl.ds(cbase + g * 16, 16)] = vals * SCALE

            @pl.when(t + _NB < _NTASK)
            def _next():
                build_and_fire(t + _NB, b)

            for kt in range(8):
                pltpu.async_copy(
                    obuf.at[b, pl.ds(kt * 1024, 1024)],
                    out_hbm.at[j, kt, it],
                    osems[b],
                )

    for b in range(_NB):
        for kt in range(8):
            pltpu.make_async_copy(
                obuf.at[b, pl.ds(0, 1024)], out_hbm.at[0, 0, 0], osems[b]
            ).wait()


def kernel(x, lut):
    x_flat = x.reshape(-1).astype(jnp.int32)
    lut_lin = _fmt(lut.T, lut.T).reshape(2 * _FH, D_MODEL)
    out5 = _emb_kernel(x_flat, lut_lin)
    out5 = out5.reshape(_NJ, 8, _NI // 128, 8, 128)
    return out5.transpose(2, 4, 0, 1, 3).reshape(_NI, _NJ, D_MODEL)


# speedup vs baseline: 3.0027x; 1.0031x over previous
import functools
import math

import jax
import jax.numpy as jnp
from jax import lax
from jax.experimental import pallas as pl
from jax.experimental.pallas import tpu as pltpu
from jax.experimental.pallas import tpu_sc as plsc

D_MODEL = 64
SCALE = math.sqrt(D_MODEL)

_info = plsc.get_sparse_core_info()
_NC, _NS, _L = _info.num_cores, _info.num_subcores, _info.num_lanes
_NW = _NC * _NS

_NI = 16384
_NJ = 50
_B = _NI * _NJ
_IB = _NI // _NW
_NQ = _IB // 128
_NTASK = _NJ * _NQ
_IDXW = _IB * _NJ
_NB = 2
_DPAD = 65

_FW = 16384
_FH = 524288
_FD = 491520


def _fmt_body(in_lo, in_hi, outb):
    lo = jnp.transpose(in_lo[...], (1, 0))
    hi = jnp.transpose(in_hi[...], (1, 0))
    outb[...] = jnp.concatenate([lo, hi], axis=1)


_fmt = pl.pallas_call(
    _fmt_body,
    out_shape=jax.ShapeDtypeStruct((_FH, 128), jnp.float32),
    grid=(_FH // _FW,),
    in_specs=[
        pl.BlockSpec((64, _FW), lambda i: (0, i)),
        pl.BlockSpec((64, _FW), lambda i: (0, i + _FD // _FW)),
    ],
    out_specs=pl.BlockSpec((_FW, 128), lambda i: (i, 0)),
)


@functools.partial(
    pl.kernel,
    out_type=jax.ShapeDtypeStruct((_NJ, 8, _NI // 128, 1024), jnp.float32),
    mesh=plsc.VectorSubcoreMesh(core_axis_name="c", subcore_axis_name="s"),
    scratch_types=[
        pltpu.VMEM((_IDXW,), jnp.int32),
        pltpu.VMEM((_NB, 128), jnp.int32),
        pltpu.VMEM((_NB, 128, D_MODEL), jnp.float32),
        pltpu.VMEM((_NB, 128, _DPAD), jnp.float32),
        pltpu.VMEM((_NB, 8192), jnp.float32),
    ]
    + [pltpu.SemaphoreType.DMA] * (2 * _NB),
    compiler_params=pltpu.CompilerParams(
        use_tc_tiling_on_sc=False, needs_layout_passes=False
    ),
)
def _emb_kernel(x_hbm, lut_hbm, out_hbm, idx_v, idxc, rows, rows65, obuf, *sems):
    gsems, osems = sems[:_NB], sems[_NB:]
    wid = lax.axis_index("s") * _NC + lax.axis_index("c")
    pltpu.sync_copy(x_hbm.at[pl.ds(wid * _IDXW, _IDXW)], idx_v)

    iota = lax.iota(jnp.int32, 16)
    iota_nj = iota * _NJ

    def jq(t):
        return lax.rem(t, _NJ), lax.div(t, _NJ)

    def build_and_fire(t, b):
        j, q = jq(t)
        base = q * (128 * _NJ) + j
        for g in range(8):
            pos = base + (g * 16 * _NJ) + iota_nj
            v = plsc.load_gather(idx_v, [pos])
            idxc[b, pl.ds(g * 16, 16)] = jnp.where(
                v >= _FH, (v - _FD) * 2 + 1, v * 2
            )
        pltpu.async_copy(lut_hbm.at[idxc.at[b]], rows.at[b], gsems[b])

    for b in range(_NB):
        build_and_fire(b, b)

    @pl.loop(0, _NTASK // _NB)
    def _task(tt):
        for b in range(_NB):
            t = tt * _NB + b
            j, q = jq(t)
            it = wid * _NQ + q
            pltpu.make_async_copy(
                lut_hbm.at[pl.ds(0, 128)], rows.at[b], gsems[b]
            ).wait()
            @pl.when(t >= _NB)
            def _drain():
                for kt in range(8):
                    pltpu.make_async_copy(
                        obuf.at[b, pl.ds(0, 1024)],
                        out_hbm.at[0, 0, 0],
                        osems[b],
                    ).wait()

            @plsc.parallel_loop(0, 128, unroll=8)
            def _restride(r):
                for sseg in range(4):
                    sl = pl.ds(sseg * 16, 16)
                    rows65[b, r, sl] = rows[b, r, sl]

            @plsc.parallel_loop(0, D_MODEL, unroll=4)
            def _col(c):
                cbase = (
                    lax.shift_left(lax.shift_right_logical(c, 3), 10)
                    + lax.shift_left(lax.bitwise_and(c, 7), 7)
                )
                colv = iota * 0 + c
                for g in range(8):
                    vals = plsc.load_gather(
                        rows65.at[b], [iota + g * 16, colv]
                    )
                    obuf[b, pl.ds(cbase + g * 16, 16)] = vals * SCALE

            @pl.when(t + _NB < _NTASK)
            def _next():
                build_and_fire(t + _NB, b)

            for kt in range(8):
                pltpu.async_copy(
                    obuf.at[b, pl.ds(kt * 1024, 1024)],
                    out_hbm.at[j, kt, it],
                    osems[b],
                )

    for b in range(_NB):
        for kt in range(8):
            pltpu.make_async_copy(
                obuf.at[b, pl.ds(0, 1024)], out_hbm.at[0, 0, 0], osems[b]
            ).wait()


def kernel(x, lut):
    x_flat = x.reshape(-1).astype(jnp.int32)
    lut_lin = _fmt(lut.T, lut.T).reshape(2 * _FH, D_MODEL)
    out5 = _emb_kernel(x_flat, lut_lin)
    out5 = out5.reshape(_NJ, 8, _NI // 128, 8, 128)
    return out5.transpose(2, 4, 0, 1, 3).reshape(_NI, _NJ, D_MODEL)
